# baseline (speedup 1.0000x reference)
import functools

import jax
import jax.numpy as jnp
from jax import lax
from jax.experimental import pallas as pl
from jax.experimental.pallas import tpu as pltpu
from jax.experimental.pallas import tpu_sc as plsc

NC, NS = 2, 16
NW = NC * NS
LANES = 16
CHUNK = 125
BM = 512
ACC_STEP = (CHUNK // 8) * 8


def _sc_mesh():
    return plsc.VectorSubcoreMesh(
        core_axis_name="c", subcore_axis_name="s",
        num_cores=NC, num_subcores=NS)


def _acc_slices(rows_per_sub):
    out = []
    off = 0
    while off < rows_per_sub:
        ln = min(ACC_STEP, rows_per_sub - off)
        out.append((off, ln))
        off += ln
    return out


def _make_deg_kernel(n_acc, n_pad, kd, d):
    rows_per_sub = n_acc // NS
    slices = _acc_slices(rows_per_sub)

    @functools.partial(
        pl.kernel,
        out_type=jax.ShapeDtypeStruct((NC, n_pad, d), jnp.float32),
        mesh=_sc_mesh(),
        scratch_types=[
            pltpu.VMEM((kd, CHUNK), jnp.int32),
            pltpu.VMEM((CHUNK, d), jnp.float32),
            pltpu.VMEM_SHARED((n_acc, d), jnp.float32),
        ],
    )
    def deg_kernel(dst_hbm, out_hbm, idx_v, ones_v, acc_sh):
        cid = lax.axis_index("c")
        sid = lax.axis_index("s")
        w = cid * NS + sid

        @pl.loop(0, CHUNK)
        def _(r):
            @pl.loop(0, d, step=LANES)
            def _(c):
                ones_v[r, pl.ds(c, LANES)] = jnp.zeros((LANES,), jnp.float32)

        for off, ln in slices:
            pltpu.sync_copy(
                ones_v.at[pl.ds(0, ln)],
                acc_sh.at[pl.ds(sid * rows_per_sub + off, ln)])

        @pl.loop(0, CHUNK)
        def _(r):
            @pl.loop(0, d, step=LANES)
            def _(c):
                ones_v[r, pl.ds(c, LANES)] = jnp.full(
                    (LANES,), 1.0, jnp.float32)

        plsc.subcore_barrier()
        pltpu.sync_copy(dst_hbm.at[pl.ds(w * kd, kd)], idx_v)

        @pl.loop(0, kd)
        def _(j):
            pltpu.sync_copy(ones_v, acc_sh.at[idx_v.at[j]], add=True)

        plsc.subcore_barrier()
        for off, ln in slices:
            pltpu.sync_copy(
                acc_sh.at[pl.ds(sid * rows_per_sub + off, ln)],
                out_hbm.at[cid, pl.ds(sid * rows_per_sub + off, ln)])

    return deg_kernel


def _make_agg_kernel(n_acc, n_pad, k0, k1, d):
    rows_per_sub = n_acc // NS
    slices = _acc_slices(rows_per_sub)
    assert k0 % 8 == 0 and k1 % 8 == 0 and rows_per_sub % 8 == 0
    kmax = max(k0, k1)

    @functools.partial(
        pl.kernel,
        out_type=jax.ShapeDtypeStruct((NC, n_pad, d), jnp.float32),
        mesh=_sc_mesh(),
        scratch_types=[
            pltpu.VMEM((kmax, CHUNK), jnp.int32),
            pltpu.VMEM((kmax, CHUNK), jnp.int32),
            pltpu.VMEM((CHUNK, d), jnp.float32),
            pltpu.VMEM((CHUNK, d), jnp.float32),
            pltpu.VMEM_SHARED((n_acc, d), jnp.float32),
            pltpu.SemaphoreType.DMA,
            pltpu.SemaphoreType.DMA,
            pltpu.SemaphoreType.DMA,
            pltpu.SemaphoreType.DMA,
        ],
    )
    def agg_kernel(table_hbm, src_hbm, dst_hbm, out_hbm,
                   si_v, di_v, rows0_v, rows1_v, acc_sh,
                   semg0, semg1, sems0, sems1):
        cid = lax.axis_index("c")
        sid = lax.axis_index("s")

        @pl.loop(0, CHUNK)
        def _(r):
            @pl.loop(0, d, step=LANES)
            def _(c):
                rows0_v[r, pl.ds(c, LANES)] = jnp.zeros((LANES,), jnp.float32)

        for off, ln in slices:
            pltpu.sync_copy(
                rows0_v.at[pl.ds(0, ln)],
                acc_sh.at[pl.ds(sid * rows_per_sub + off, ln)])

        plsc.subcore_barrier()

        def run(k, base):
            pltpu.sync_copy(src_hbm.at[pl.ds(base, k)], si_v.at[pl.ds(0, k)])
            pltpu.sync_copy(dst_hbm.at[pl.ds(base, k)], di_v.at[pl.ds(0, k)])
            pltpu.async_copy(table_hbm.at[si_v.at[0]], rows0_v, semg0)
            pltpu.async_copy(table_hbm.at[si_v.at[1]], rows1_v, semg1)

            @pl.loop(0, k, step=2)
            def _(j):
                pltpu.make_async_copy(
                    table_hbm.at[si_v.at[j]], rows0_v, semg0).wait()
                pltpu.async_copy(rows0_v, acc_sh.at[di_v.at[j]], sems0,
                                 add=True)
                pltpu.make_async_copy(
                    table_hbm.at[si_v.at[j + 1]], rows1_v, semg1).wait()
                pltpu.async_copy(rows1_v, acc_sh.at[di_v.at[j + 1]], sems1,
                                 add=True)
                pltpu.make_async_copy(
                    rows0_v, acc_sh.at[di_v.at[j]], sems0).wait()

                @pl.when(j + 2 < k)
                def _():
                    pltpu.async_copy(
                        table_hbm.at[si_v.at[j + 2]], rows0_v, semg0)

                pltpu.make_async_copy(
                    rows1_v, acc_sh.at[di_v.at[j + 1]], sems1).wait()

                @pl.when(j + 3 < k)
                def _():
                    pltpu.async_copy(
                        table_hbm.at[si_v.at[j + 3]], rows1_v, semg1)

        if k0 == k1:
            run(k0, (cid * NS + sid) * k0)
        else:
            @pl.when(cid == 0)
            def _():
                run(k0, sid * k0)

            @pl.when(cid == 1)
            def _():
                run(k1, NS * k0 + sid * k1)

        plsc.subcore_barrier()
        for off, ln in slices:
            pltpu.sync_copy(
                acc_sh.at[pl.ds(sid * rows_per_sub + off, ln)],
                out_hbm.at[cid, pl.ds(sid * rows_per_sub + off, ln)])

    return agg_kernel


def _tc_matmul(x, W, n_pad):
    n, d_in = x.shape
    d_o = W.shape[1]

    def body(x_ref, w_ref, o_ref):
        o_ref[...] = jnp.dot(x_ref[...], w_ref[...],
                             preferred_element_type=jnp.float32)

    return pl.pallas_call(
        body,
        grid=(n_pad // BM,),
        in_specs=[pl.BlockSpec((BM, d_in), lambda i: (i, 0)),
                  pl.BlockSpec((d_in, d_o), lambda i: (0, 0))],
        out_specs=pl.BlockSpec((BM, d_o), lambda i: (i, 0)),
        out_shape=jax.ShapeDtypeStruct((n_pad, d_o), jnp.float32),
    )(x, W)


def _tc_scale(h, degc):
    n_pad, d = h.shape

    def body(h_ref, dg_ref, ht_ref, di_ref):
        dinv = lax.rsqrt(dg_ref[0] + dg_ref[1] + 1.0)
        di_ref[...] = dinv
        ht_ref[...] = dinv * h_ref[...]

    return pl.pallas_call(
        body,
        grid=(n_pad // BM,),
        in_specs=[pl.BlockSpec((BM, d), lambda i: (i, 0)),
                  pl.BlockSpec((NC, BM, d), lambda i: (0, i, 0))],
        out_specs=[pl.BlockSpec((BM, d), lambda i: (i, 0)),
                   pl.BlockSpec((BM, d), lambda i: (i, 0))],
        out_shape=[jax.ShapeDtypeStruct((n_pad, d), jnp.float32),
                   jax.ShapeDtypeStruct((n_pad, d), jnp.float32)],
    )(h, degc)


def _tc_mid(agg, ht, dinv, b1):
    n_pad, d = ht.shape

    def body(ag_ref, ht_ref, di_ref, b_ref, o_ref):
        dinv = di_ref[...]
        z = dinv * (ag_ref[0] + ag_ref[1] + ht_ref[...]) + b_ref[...]
        z = jnp.maximum(z, 0.0)
        o_ref[...] = dinv * z

    return pl.pallas_call(
        body,
        grid=(n_pad // BM,),
        in_specs=[pl.BlockSpec((NC, BM, d), lambda i: (0, i, 0)),
                  pl.BlockSpec((BM, d), lambda i: (i, 0)),
                  pl.BlockSpec((BM, d), lambda i: (i, 0)),
                  pl.BlockSpec((1, d), lambda i: (0, 0))],
        out_specs=pl.BlockSpec((BM, d), lambda i: (i, 0)),
        out_shape=jax.ShapeDtypeStruct((n_pad, d), jnp.float32),
    )(agg, ht, dinv, b1)


def _tc_out(agg, ht, dinv, W2, b2, n_out):
    n_pad, d = ht.shape
    d_o = W2.shape[1]

    def body(ag_ref, ht_ref, di_ref, w_ref, b_ref, o_ref):
        pre = di_ref[...] * (ag_ref[0] + ag_ref[1] + ht_ref[...])
        acc = jnp.dot(pre, w_ref[...], preferred_element_type=jnp.float32)
        o_ref[...] = jnp.maximum(acc + b_ref[...], 0.0)

    return pl.pallas_call(
        body,
        grid=(n_pad // BM,),
        in_specs=[pl.BlockSpec((NC, BM, d), lambda i: (0, i, 0)),
                  pl.BlockSpec((BM, d), lambda i: (i, 0)),
                  pl.BlockSpec((BM, d), lambda i: (i, 0)),
                  pl.BlockSpec((d, d_o), lambda i: (0, 0)),
                  pl.BlockSpec((1, d_o), lambda i: (0, 0))],
        out_specs=pl.BlockSpec((BM, d_o), lambda i: (i, 0)),
        out_shape=jax.ShapeDtypeStruct((n_out, d_o), jnp.float32),
    )(agg, ht, dinv, W2, b2)


def kernel(x, edge_index, W1, b1, W2, b2):
    N, d_in = x.shape
    d_hid = W1.shape[1]
    d_out = W2.shape[1]
    E = edge_index.shape[1]

    kt = -(-E // (NS * CHUNK))
    kt = -(-kt // 16) * 16
    k0 = kt // 2
    k1 = kt - k0
    n_chunks = NS * kt
    e_pad = n_chunks * CHUNK
    assert n_chunks % NW == 0
    kd = n_chunks // NW

    n_acc = -(-(N + LANES) // (NS * 8)) * (NS * 8)
    n_pad = -(-max(n_acc, BM) // BM) * BM

    src = edge_index[0].astype(jnp.int32)
    dst = edge_index[1].astype(jnp.int32)
    if e_pad > E:
        fill = N + (jnp.arange(e_pad - E, dtype=jnp.int32) % (n_acc - N))
        src = jnp.concatenate([src, fill])
        dst = jnp.concatenate([dst, fill])
    src_r = src.reshape(n_chunks, CHUNK)
    dst_r = dst.reshape(n_chunks, CHUNK)
    b1r = b1.reshape(1, d_hid)
    b2r = b2.reshape(1, d_out)

    deg_fn = _make_deg_kernel(n_acc, n_pad, kd, d_hid)
    agg_fn = _make_agg_kernel(n_acc, n_pad, k0, k1, d_hid)

    degc = deg_fn(dst_r)
    h1 = _tc_matmul(x, W1, n_pad)
    ht1, dinv = _tc_scale(h1, degc)
    agg1 = agg_fn(ht1, src_r, dst_r)
    htz = _tc_mid(agg1, ht1, dinv, b1r)
    agg2 = agg_fn(htz, src_r, dst_r)
    return _tc_out(agg2, htz, dinv, W2, b2r, N)


# speedup vs baseline: 1.0567x; 1.0567x over previous
import functools

import jax
import jax.numpy as jnp
from jax import lax
from jax.experimental import pallas as pl
from jax.experimental.pallas import tpu as pltpu
from jax.experimental.pallas import tpu_sc as plsc

NC, NS = 2, 16
NW = NC * NS
LANES = 16
CHUNK = 125
BM = 512
ACC_STEP = (CHUNK // 8) * 8


def _sc_mesh():
    return plsc.VectorSubcoreMesh(
        core_axis_name="c", subcore_axis_name="s",
        num_cores=NC, num_subcores=NS)


def _acc_slices(rows_per_sub):
    out = []
    off = 0
    while off < rows_per_sub:
        ln = min(ACC_STEP, rows_per_sub - off)
        out.append((off, ln))
        off += ln
    return out


def _make_deg_kernel(n_acc, n_pad, kd, d):
    rows_per_sub = n_acc // NS
    slices = _acc_slices(rows_per_sub)

    @functools.partial(
        pl.kernel,
        out_type=jax.ShapeDtypeStruct((NC, n_pad, d), jnp.float32),
        mesh=_sc_mesh(),
        scratch_types=[
            pltpu.VMEM((kd, CHUNK), jnp.int32),
            pltpu.VMEM((CHUNK, d), jnp.float32),
            pltpu.VMEM_SHARED((n_acc, d), jnp.float32),
        ],
    )
    def deg_kernel(dst_hbm, out_hbm, idx_v, ones_v, acc_sh):
        cid = lax.axis_index("c")
        sid = lax.axis_index("s")
        w = cid * NS + sid

        @pl.loop(0, CHUNK)
        def _(r):
            @pl.loop(0, d, step=LANES)
            def _(c):
                ones_v[r, pl.ds(c, LANES)] = jnp.zeros((LANES,), jnp.float32)

        for off, ln in slices:
            pltpu.sync_copy(
                ones_v.at[pl.ds(0, ln)],
                acc_sh.at[pl.ds(sid * rows_per_sub + off, ln)])

        @pl.loop(0, CHUNK)
        def _(r):
            @pl.loop(0, d, step=LANES)
            def _(c):
                ones_v[r, pl.ds(c, LANES)] = jnp.full(
                    (LANES,), 1.0, jnp.float32)

        plsc.subcore_barrier()
        pltpu.sync_copy(dst_hbm.at[pl.ds(w * kd, kd)], idx_v)

        @pl.loop(0, kd)
        def _(j):
            pltpu.sync_copy(ones_v, acc_sh.at[idx_v.at[j]], add=True)

        plsc.subcore_barrier()
        for off, ln in slices:
            pltpu.sync_copy(
                acc_sh.at[pl.ds(sid * rows_per_sub + off, ln)],
                out_hbm.at[cid, pl.ds(sid * rows_per_sub + off, ln)])

    return deg_kernel


def _make_agg_kernel(n_acc, n_pad, k0, k1, d):
    rows_per_sub = n_acc // NS
    slices = _acc_slices(rows_per_sub)
    assert k0 % 8 == 0 and k1 % 8 == 0 and rows_per_sub % 8 == 0
    kmax = max(k0, k1)

    @functools.partial(
        pl.kernel,
        out_type=jax.ShapeDtypeStruct((NC, n_pad, d), jnp.float32),
        mesh=_sc_mesh(),
        scratch_types=[
            pltpu.VMEM((kmax, CHUNK), jnp.int32),
            pltpu.VMEM((kmax, CHUNK), jnp.int32),
            pltpu.VMEM((CHUNK, d), jnp.float32),
            pltpu.VMEM((CHUNK, d), jnp.float32),
            pltpu.VMEM_SHARED((n_acc, d), jnp.float32),
            pltpu.SemaphoreType.DMA,
            pltpu.SemaphoreType.DMA,
            pltpu.SemaphoreType.DMA,
            pltpu.SemaphoreType.DMA,
        ],
    )
    def agg_kernel(table_hbm, src_hbm, dst_hbm, out_hbm,
                   si_v, di_v, rows0_v, rows1_v, acc_sh,
                   semg0, semg1, sems0, sems1):
        cid = lax.axis_index("c")
        sid = lax.axis_index("s")

        @pl.loop(0, CHUNK)
        def _(r):
            @pl.loop(0, d, step=LANES)
            def _(c):
                rows0_v[r, pl.ds(c, LANES)] = jnp.zeros((LANES,), jnp.float32)

        for off, ln in slices:
            pltpu.sync_copy(
                rows0_v.at[pl.ds(0, ln)],
                acc_sh.at[pl.ds(sid * rows_per_sub + off, ln)])

        plsc.subcore_barrier()

        def run(k, base):
            pltpu.sync_copy(src_hbm.at[pl.ds(base, k)], si_v.at[pl.ds(0, k)])
            pltpu.sync_copy(dst_hbm.at[pl.ds(base, k)], di_v.at[pl.ds(0, k)])
            pltpu.async_copy(table_hbm.at[si_v.at[0]], rows0_v, semg0)

            @pl.loop(0, k, step=2)
            def _(j):
                pltpu.make_async_copy(
                    table_hbm.at[si_v.at[j]], rows0_v, semg0).wait()
                pltpu.async_copy(table_hbm.at[si_v.at[j + 1]], rows1_v, semg1)
                pltpu.sync_copy(rows0_v, acc_sh.at[di_v.at[j]], add=True)
                pltpu.make_async_copy(
                    table_hbm.at[si_v.at[j + 1]], rows1_v, semg1).wait()

                @pl.when(j + 2 < k)
                def _():
                    pltpu.async_copy(
                        table_hbm.at[si_v.at[j + 2]], rows0_v, semg0)

                pltpu.sync_copy(rows1_v, acc_sh.at[di_v.at[j + 1]], add=True)

        if k0 == k1:
            run(k0, (cid * NS + sid) * k0)
        else:
            @pl.when(cid == 0)
            def _():
                run(k0, sid * k0)

            @pl.when(cid == 1)
            def _():
                run(k1, NS * k0 + sid * k1)

        plsc.subcore_barrier()
        for off, ln in slices:
            pltpu.sync_copy(
                acc_sh.at[pl.ds(sid * rows_per_sub + off, ln)],
                out_hbm.at[cid, pl.ds(sid * rows_per_sub + off, ln)])

    return agg_kernel


def _tc_matmul(x, W, n_pad):
    n, d_in = x.shape
    d_o = W.shape[1]

    def body(x_ref, w_ref, o_ref):
        o_ref[...] = jnp.dot(x_ref[...], w_ref[...],
                             preferred_element_type=jnp.float32)

    return pl.pallas_call(
        body,
        grid=(n_pad // BM,),
        in_specs=[pl.BlockSpec((BM, d_in), lambda i: (i, 0)),
                  pl.BlockSpec((d_in, d_o), lambda i: (0, 0))],
        out_specs=pl.BlockSpec((BM, d_o), lambda i: (i, 0)),
        out_shape=jax.ShapeDtypeStruct((n_pad, d_o), jnp.float32),
    )(x, W)


def _tc_scale(h, degc):
    n_pad, d = h.shape

    def body(h_ref, dg_ref, ht_ref, di_ref):
        dinv = lax.rsqrt(dg_ref[0] + dg_ref[1] + 1.0)
        di_ref[...] = dinv
        ht_ref[...] = dinv * h_ref[...]

    return pl.pallas_call(
        body,
        grid=(n_pad // BM,),
        in_specs=[pl.BlockSpec((BM, d), lambda i: (i, 0)),
                  pl.BlockSpec((NC, BM, d), lambda i: (0, i, 0))],
        out_specs=[pl.BlockSpec((BM, d), lambda i: (i, 0)),
                   pl.BlockSpec((BM, d), lambda i: (i, 0))],
        out_shape=[jax.ShapeDtypeStruct((n_pad, d), jnp.float32),
                   jax.ShapeDtypeStruct((n_pad, d), jnp.float32)],
    )(h, degc)


def _tc_mid(agg, ht, dinv, b1):
    n_pad, d = ht.shape

    def body(ag_ref, ht_ref, di_ref, b_ref, o_ref):
        dinv = di_ref[...]
        z = dinv * (ag_ref[0] + ag_ref[1] + ht_ref[...]) + b_ref[...]
        z = jnp.maximum(z, 0.0)
        o_ref[...] = dinv * z

    return pl.pallas_call(
        body,
        grid=(n_pad // BM,),
        in_specs=[pl.BlockSpec((NC, BM, d), lambda i: (0, i, 0)),
                  pl.BlockSpec((BM, d), lambda i: (i, 0)),
                  pl.BlockSpec((BM, d), lambda i: (i, 0)),
                  pl.BlockSpec((1, d), lambda i: (0, 0))],
        out_specs=pl.BlockSpec((BM, d), lambda i: (i, 0)),
        out_shape=jax.ShapeDtypeStruct((n_pad, d), jnp.float32),
    )(agg, ht, dinv, b1)


def _tc_out(agg, ht, dinv, W2, b2, n_out):
    n_pad, d = ht.shape
    d_o = W2.shape[1]

    def body(ag_ref, ht_ref, di_ref, w_ref, b_ref, o_ref):
        pre = di_ref[...] * (ag_ref[0] + ag_ref[1] + ht_ref[...])
        acc = jnp.dot(pre, w_ref[...], preferred_element_type=jnp.float32)
        o_ref[...] = jnp.maximum(acc + b_ref[...], 0.0)

    return pl.pallas_call(
        body,
        grid=(n_pad // BM,),
        in_specs=[pl.BlockSpec((NC, BM, d), lambda i: (0, i, 0)),
                  pl.BlockSpec((BM, d), lambda i: (i, 0)),
                  pl.BlockSpec((BM, d), lambda i: (i, 0)),
                  pl.BlockSpec((d, d_o), lambda i: (0, 0)),
                  pl.BlockSpec((1, d_o), lambda i: (0, 0))],
        out_specs=pl.BlockSpec((BM, d_o), lambda i: (i, 0)),
        out_shape=jax.ShapeDtypeStruct((n_out, d_o), jnp.float32),
    )(agg, ht, dinv, W2, b2)


def kernel(x, edge_index, W1, b1, W2, b2):
    N, d_in = x.shape
    d_hid = W1.shape[1]
    d_out = W2.shape[1]
    E = edge_index.shape[1]

    kt = -(-E // (NS * CHUNK))
    kt = -(-kt // 16) * 16
    k0 = kt // 2
    k1 = kt - k0
    n_chunks = NS * kt
    e_pad = n_chunks * CHUNK
    assert n_chunks % NW == 0
    kd = n_chunks // NW

    n_acc = -(-(N + LANES) // (NS * 8)) * (NS * 8)
    n_pad = -(-max(n_acc, BM) // BM) * BM

    src = edge_index[0].astype(jnp.int32)
    dst = edge_index[1].astype(jnp.int32)
    if e_pad > E:
        fill = N + (jnp.arange(e_pad - E, dtype=jnp.int32) % (n_acc - N))
        src = jnp.concatenate([src, fill])
        dst = jnp.concatenate([dst, fill])
    src_r = src.reshape(n_chunks, CHUNK)
    dst_r = dst.reshape(n_chunks, CHUNK)
    b1r = b1.reshape(1, d_hid)
    b2r = b2.reshape(1, d_out)

    deg_fn = _make_deg_kernel(n_acc, n_pad, kd, d_hid)
    agg_fn = _make_agg_kernel(n_acc, n_pad, k0, k1, d_hid)

    degc = deg_fn(dst_r)
    h1 = _tc_matmul(x, W1, n_pad)
    ht1, dinv = _tc_scale(h1, degc)
    agg1 = agg_fn(ht1, src_r, dst_r)
    htz = _tc_mid(agg1, ht1, dinv, b1r)
    agg2 = agg_fn(htz, src_r, dst_r)
    return _tc_out(agg2, htz, dinv, W2, b2r, N)


# speedup vs baseline: 1.1238x; 1.0635x over previous
import functools

import jax
import jax.numpy as jnp
from jax import lax
from jax.experimental import pallas as pl
from jax.experimental.pallas import tpu as pltpu
from jax.experimental.pallas import tpu_sc as plsc

NC, NS = 2, 16
NW = NC * NS
LANES = 16
CHUNK = 125
BM = 1024
ACC_STEP = (CHUNK // 8) * 8


def _sc_mesh():
    return plsc.VectorSubcoreMesh(
        core_axis_name="c", subcore_axis_name="s",
        num_cores=NC, num_subcores=NS)


def _acc_slices(rows_per_sub):
    out = []
    off = 0
    while off < rows_per_sub:
        ln = min(ACC_STEP, rows_per_sub - off)
        out.append((off, ln))
        off += ln
    return out


def _make_deg_kernel(n_acc, n_pad, kd, d):
    rows_per_sub = n_acc // NS
    slices = _acc_slices(rows_per_sub)

    @functools.partial(
        pl.kernel,
        out_type=jax.ShapeDtypeStruct((NC, n_pad, d), jnp.float32),
        mesh=_sc_mesh(),
        scratch_types=[
            pltpu.VMEM((kd, CHUNK), jnp.int32),
            pltpu.VMEM((CHUNK, d), jnp.float32),
            pltpu.VMEM_SHARED((n_acc, d), jnp.float32),
        ],
    )
    def deg_kernel(dst_hbm, out_hbm, idx_v, ones_v, acc_sh):
        cid = lax.axis_index("c")
        sid = lax.axis_index("s")
        w = cid * NS + sid

        @pl.loop(0, CHUNK)
        def _(r):
            @pl.loop(0, d, step=LANES)
            def _(c):
                ones_v[r, pl.ds(c, LANES)] = jnp.zeros((LANES,), jnp.float32)

        for off, ln in slices:
            pltpu.sync_copy(
                ones_v.at[pl.ds(0, ln)],
                acc_sh.at[pl.ds(sid * rows_per_sub + off, ln)])

        @pl.loop(0, CHUNK)
        def _(r):
            @pl.loop(0, d, step=LANES)
            def _(c):
                ones_v[r, pl.ds(c, LANES)] = jnp.full(
                    (LANES,), 1.0, jnp.float32)

        plsc.subcore_barrier()
        pltpu.sync_copy(dst_hbm.at[pl.ds(w * kd, kd)], idx_v)

        @pl.loop(0, kd)
        def _(j):
            pltpu.sync_copy(ones_v, acc_sh.at[idx_v.at[j]], add=True)

        plsc.subcore_barrier()
        for off, ln in slices:
            pltpu.sync_copy(
                acc_sh.at[pl.ds(sid * rows_per_sub + off, ln)],
                out_hbm.at[cid, pl.ds(sid * rows_per_sub + off, ln)])

    return deg_kernel


def _make_agg_kernel(n_acc, n_pad, k0, k1, d):
    rows_per_sub = n_acc // NS
    slices = _acc_slices(rows_per_sub)
    assert k0 % 8 == 0 and k1 % 8 == 0 and rows_per_sub % 8 == 0
    kmax = max(k0, k1)

    @functools.partial(
        pl.kernel,
        out_type=jax.ShapeDtypeStruct((NC, n_pad, d), jnp.float32),
        mesh=_sc_mesh(),
        scratch_types=[
            pltpu.VMEM((kmax, CHUNK), jnp.int32),
            pltpu.VMEM((kmax, CHUNK), jnp.int32),
            pltpu.VMEM((CHUNK, d), jnp.float32),
            pltpu.VMEM((CHUNK, d), jnp.float32),
            pltpu.VMEM_SHARED((n_acc, d), jnp.float32),
            pltpu.SemaphoreType.DMA,
            pltpu.SemaphoreType.DMA,
            pltpu.SemaphoreType.DMA,
            pltpu.SemaphoreType.DMA,
        ],
    )
    def agg_kernel(table_hbm, src_hbm, dst_hbm, out_hbm,
                   si_v, di_v, rows0_v, rows1_v, acc_sh,
                   semg0, semg1, sems0, sems1):
        cid = lax.axis_index("c")
        sid = lax.axis_index("s")

        @pl.loop(0, CHUNK)
        def _(r):
            @pl.loop(0, d, step=LANES)
            def _(c):
                rows0_v[r, pl.ds(c, LANES)] = jnp.zeros((LANES,), jnp.float32)

        for off, ln in slices:
            pltpu.sync_copy(
                rows0_v.at[pl.ds(0, ln)],
                acc_sh.at[pl.ds(sid * rows_per_sub + off, ln)])

        plsc.subcore_barrier()

        def run(k, base):
            pltpu.sync_copy(src_hbm.at[pl.ds(base, k)], si_v.at[pl.ds(0, k)])
            pltpu.sync_copy(dst_hbm.at[pl.ds(base, k)], di_v.at[pl.ds(0, k)])
            pltpu.async_copy(table_hbm.at[si_v.at[0]], rows0_v, semg0)

            @pl.loop(0, k, step=2)
            def _(j):
                pltpu.make_async_copy(
                    table_hbm.at[si_v.at[j]], rows0_v, semg0).wait()
                pltpu.async_copy(table_hbm.at[si_v.at[j + 1]], rows1_v, semg1)
                pltpu.sync_copy(rows0_v, acc_sh.at[di_v.at[j]], add=True)
                pltpu.make_async_copy(
                    table_hbm.at[si_v.at[j + 1]], rows1_v, semg1).wait()

                @pl.when(j + 2 < k)
                def _():
                    pltpu.async_copy(
                        table_hbm.at[si_v.at[j + 2]], rows0_v, semg0)

                pltpu.sync_copy(rows1_v, acc_sh.at[di_v.at[j + 1]], add=True)

        if k0 == k1:
            run(k0, (cid * NS + sid) * k0)
        else:
            @pl.when(cid == 0)
            def _():
                run(k0, sid * k0)

            @pl.when(cid == 1)
            def _():
                run(k1, NS * k0 + sid * k1)

        plsc.subcore_barrier()
        for off, ln in slices:
            pltpu.sync_copy(
                acc_sh.at[pl.ds(sid * rows_per_sub + off, ln)],
                out_hbm.at[cid, pl.ds(sid * rows_per_sub + off, ln)])

    return agg_kernel


def _tc_matmul(x, W, n_pad):
    n, d_in = x.shape
    d_o = W.shape[1]

    def body(x_ref, w_ref, o_ref):
        o_ref[...] = jnp.dot(x_ref[...], w_ref[...],
                             preferred_element_type=jnp.float32)

    return pl.pallas_call(
        body,
        grid=(n_pad // BM,),
        in_specs=[pl.BlockSpec((BM, d_in), lambda i: (i, 0)),
                  pl.BlockSpec((d_in, d_o), lambda i: (0, 0))],
        out_specs=pl.BlockSpec((BM, d_o), lambda i: (i, 0)),
        out_shape=jax.ShapeDtypeStruct((n_pad, d_o), jnp.float32),
    )(x, W)


def _tc_scale(h, degc):
    n_pad, d = h.shape

    def body(h_ref, dg_ref, ht_ref, di_ref):
        dinv = lax.rsqrt(dg_ref[0] + dg_ref[1] + 1.0)
        di_ref[...] = dinv
        ht_ref[...] = dinv * h_ref[...]

    return pl.pallas_call(
        body,
        grid=(n_pad // BM,),
        in_specs=[pl.BlockSpec((BM, d), lambda i: (i, 0)),
                  pl.BlockSpec((NC, BM, d), lambda i: (0, i, 0))],
        out_specs=[pl.BlockSpec((BM, d), lambda i: (i, 0)),
                   pl.BlockSpec((BM, d), lambda i: (i, 0))],
        out_shape=[jax.ShapeDtypeStruct((n_pad, d), jnp.float32),
                   jax.ShapeDtypeStruct((n_pad, d), jnp.float32)],
    )(h, degc)


def _tc_mid(agg, ht, dinv, b1):
    n_pad, d = ht.shape

    def body(ag_ref, ht_ref, di_ref, b_ref, o_ref):
        dinv = di_ref[...]
        z = dinv * (ag_ref[0] + ag_ref[1] + ht_ref[...]) + b_ref[...]
        z = jnp.maximum(z, 0.0)
        o_ref[...] = dinv * z

    return pl.pallas_call(
        body,
        grid=(n_pad // BM,),
        in_specs=[pl.BlockSpec((NC, BM, d), lambda i: (0, i, 0)),
                  pl.BlockSpec((BM, d), lambda i: (i, 0)),
                  pl.BlockSpec((BM, d), lambda i: (i, 0)),
                  pl.BlockSpec((1, d), lambda i: (0, 0))],
        out_specs=pl.BlockSpec((BM, d), lambda i: (i, 0)),
        out_shape=jax.ShapeDtypeStruct((n_pad, d), jnp.float32),
    )(agg, ht, dinv, b1)


def _tc_out(agg, ht, dinv, W2, b2, n_out):
    n_pad, d = ht.shape
    d_o = W2.shape[1]

    def body(ag_ref, ht_ref, di_ref, w_ref, b_ref, o_ref):
        pre = di_ref[...] * (ag_ref[0] + ag_ref[1] + ht_ref[...])
        acc = jnp.dot(pre, w_ref[...], preferred_element_type=jnp.float32)
        o_ref[...] = jnp.maximum(acc + b_ref[...], 0.0)

    return pl.pallas_call(
        body,
        grid=(n_pad // BM,),
        in_specs=[pl.BlockSpec((NC, BM, d), lambda i: (0, i, 0)),
                  pl.BlockSpec((BM, d), lambda i: (i, 0)),
                  pl.BlockSpec((BM, d), lambda i: (i, 0)),
                  pl.BlockSpec((d, d_o), lambda i: (0, 0)),
                  pl.BlockSpec((1, d_o), lambda i: (0, 0))],
        out_specs=pl.BlockSpec((BM, d_o), lambda i: (i, 0)),
        out_shape=jax.ShapeDtypeStruct((n_out, d_o), jnp.float32),
    )(agg, ht, dinv, W2, b2)


def kernel(x, edge_index, W1, b1, W2, b2):
    N, d_in = x.shape
    d_hid = W1.shape[1]
    d_out = W2.shape[1]
    E = edge_index.shape[1]

    kt = -(-E // (NS * CHUNK))
    kt = -(-kt // 16) * 16
    k0 = kt // 2
    k1 = kt - k0
    n_chunks = NS * kt
    e_pad = n_chunks * CHUNK
    assert n_chunks % NW == 0
    kd = n_chunks // NW

    n_acc = -(-(N + LANES) // (NS * 8)) * (NS * 8)
    n_pad = -(-max(n_acc, BM) // BM) * BM

    src = edge_index[0].astype(jnp.int32)
    dst = edge_index[1].astype(jnp.int32)
    if e_pad > E:
        fill = N + (jnp.arange(e_pad - E, dtype=jnp.int32) % (n_acc - N))
        src = jnp.concatenate([src, fill])
        dst = jnp.concatenate([dst, fill])
    src_r = src.reshape(n_chunks, CHUNK)
    dst_r = dst.reshape(n_chunks, CHUNK)
    b1r = b1.reshape(1, d_hid)
    b2r = b2.reshape(1, d_out)

    deg_fn = _make_deg_kernel(n_acc, n_pad, kd, d_hid)
    agg_fn = _make_agg_kernel(n_acc, n_pad, k0, k1, d_hid)

    degc = deg_fn(dst_r)
    h1 = _tc_matmul(x, W1, n_pad)
    ht1, dinv = _tc_scale(h1, degc)
    agg1 = agg_fn(ht1, src_r, dst_r)
    htz = _tc_mid(agg1, ht1, dinv, b1r)
    agg2 = agg_fn(htz, src_r, dst_r)
    return _tc_out(agg2, htz, dinv, W2, b2r, N)


# speedup vs baseline: 1.1392x; 1.0136x over previous
import functools

import jax
import jax.numpy as jnp
from jax import lax
from jax.experimental import pallas as pl
from jax.experimental.pallas import tpu as pltpu
from jax.experimental.pallas import tpu_sc as plsc

NC, NS = 2, 16
NW = NC * NS
LANES = 16
CHUNK = 125
BM = 2048
ACC_STEP = (CHUNK // 8) * 8


def _sc_mesh():
    return plsc.VectorSubcoreMesh(
        core_axis_name="c", subcore_axis_name="s",
        num_cores=NC, num_subcores=NS)


def _acc_slices(rows_per_sub):
    out = []
    off = 0
    while off < rows_per_sub:
        ln = min(ACC_STEP, rows_per_sub - off)
        out.append((off, ln))
        off += ln
    return out


def _make_deg_kernel(n_acc, n_pad, kd, d):
    rows_per_sub = n_acc // NS
    slices = _acc_slices(rows_per_sub)

    @functools.partial(
        pl.kernel,
        out_type=jax.ShapeDtypeStruct((NC, n_pad, d), jnp.float32),
        mesh=_sc_mesh(),
        scratch_types=[
            pltpu.VMEM((kd, CHUNK), jnp.int32),
            pltpu.VMEM((CHUNK, d), jnp.float32),
            pltpu.VMEM_SHARED((n_acc, d), jnp.float32),
        ],
    )
    def deg_kernel(dst_hbm, out_hbm, idx_v, ones_v, acc_sh):
        cid = lax.axis_index("c")
        sid = lax.axis_index("s")
        w = cid * NS + sid

        @pl.loop(0, CHUNK)
        def _(r):
            @pl.loop(0, d, step=LANES)
            def _(c):
                ones_v[r, pl.ds(c, LANES)] = jnp.zeros((LANES,), jnp.float32)

        for off, ln in slices:
            pltpu.sync_copy(
                ones_v.at[pl.ds(0, ln)],
                acc_sh.at[pl.ds(sid * rows_per_sub + off, ln)])

        @pl.loop(0, CHUNK)
        def _(r):
            @pl.loop(0, d, step=LANES)
            def _(c):
                ones_v[r, pl.ds(c, LANES)] = jnp.full(
                    (LANES,), 1.0, jnp.float32)

        plsc.subcore_barrier()
        pltpu.sync_copy(dst_hbm.at[pl.ds(w * kd, kd)], idx_v)

        @pl.loop(0, kd)
        def _(j):
            pltpu.sync_copy(ones_v, acc_sh.at[idx_v.at[j]], add=True)

        plsc.subcore_barrier()
        for off, ln in slices:
            pltpu.sync_copy(
                acc_sh.at[pl.ds(sid * rows_per_sub + off, ln)],
                out_hbm.at[cid, pl.ds(sid * rows_per_sub + off, ln)])

    return deg_kernel


def _make_agg_kernel(n_acc, n_pad, k0, k1, d):
    rows_per_sub = n_acc // NS
    slices = _acc_slices(rows_per_sub)
    assert k0 % 8 == 0 and k1 % 8 == 0 and rows_per_sub % 8 == 0
    kmax = max(k0, k1)

    @functools.partial(
        pl.kernel,
        out_type=jax.ShapeDtypeStruct((NC, n_pad, d), jnp.float32),
        mesh=_sc_mesh(),
        scratch_types=[
            pltpu.VMEM((kmax, CHUNK), jnp.int32),
            pltpu.VMEM((kmax, CHUNK), jnp.int32),
            pltpu.VMEM((CHUNK, d), jnp.float32),
            pltpu.VMEM((CHUNK, d), jnp.float32),
            pltpu.VMEM_SHARED((n_acc, d), jnp.float32),
            pltpu.SemaphoreType.DMA,
            pltpu.SemaphoreType.DMA,
        ],
    )
    def agg_kernel(table_hbm, src_hbm, dst_hbm, out_hbm,
                   si_v, di_v, rows0_v, rows1_v, acc_sh, semg0, semg1):
        cid = lax.axis_index("c")
        sid = lax.axis_index("s")

        @pl.loop(0, CHUNK)
        def _(r):
            @pl.loop(0, d, step=LANES)
            def _(c):
                rows0_v[r, pl.ds(c, LANES)] = jnp.zeros((LANES,), jnp.float32)

        for off, ln in slices:
            pltpu.sync_copy(
                rows0_v.at[pl.ds(0, ln)],
                acc_sh.at[pl.ds(sid * rows_per_sub + off, ln)])

        plsc.subcore_barrier()

        def run(k, base):
            pltpu.sync_copy(src_hbm.at[pl.ds(base, k)], si_v.at[pl.ds(0, k)])
            pltpu.sync_copy(dst_hbm.at[pl.ds(base, k)], di_v.at[pl.ds(0, k)])
            pltpu.async_copy(table_hbm.at[si_v.at[0]], rows0_v, semg0)

            @pl.loop(0, k, step=2)
            def _(j):
                pltpu.make_async_copy(
                    table_hbm.at[si_v.at[j]], rows0_v, semg0).wait()
                pltpu.async_copy(table_hbm.at[si_v.at[j + 1]], rows1_v, semg1)
                pltpu.sync_copy(rows0_v, acc_sh.at[di_v.at[j]], add=True)
                pltpu.make_async_copy(
                    table_hbm.at[si_v.at[j + 1]], rows1_v, semg1).wait()

                @pl.when(j + 2 < k)
                def _():
                    pltpu.async_copy(
                        table_hbm.at[si_v.at[j + 2]], rows0_v, semg0)

                pltpu.sync_copy(rows1_v, acc_sh.at[di_v.at[j + 1]], add=True)

        if k0 == k1:
            run(k0, (cid * NS + sid) * k0)
        else:
            @pl.when(cid == 0)
            def _():
                run(k0, sid * k0)

            @pl.when(cid == 1)
            def _():
                run(k1, NS * k0 + sid * k1)

        plsc.subcore_barrier()
        for off, ln in slices:
            pltpu.sync_copy(
                acc_sh.at[pl.ds(sid * rows_per_sub + off, ln)],
                out_hbm.at[cid, pl.ds(sid * rows_per_sub + off, ln)])

    return agg_kernel


def _tc_matmul(x, W, n_pad):
    n, d_in = x.shape
    d_o = W.shape[1]

    def body(x_ref, w_ref, o_ref):
        o_ref[...] = jnp.dot(x_ref[...], w_ref[...],
                             preferred_element_type=jnp.float32)

    return pl.pallas_call(
        body,
        grid=(n_pad // BM,),
        in_specs=[pl.BlockSpec((BM, d_in), lambda i: (i, 0)),
                  pl.BlockSpec((d_in, d_o), lambda i: (0, 0))],
        out_specs=pl.BlockSpec((BM, d_o), lambda i: (i, 0)),
        out_shape=jax.ShapeDtypeStruct((n_pad, d_o), jnp.float32),
    )(x, W)


def _tc_scale(h, degc):
    n_pad, d = h.shape

    def body(h_ref, dg_ref, ht_ref, di_ref):
        dinv = lax.rsqrt(dg_ref[0] + dg_ref[1] + 1.0)
        di_ref[...] = dinv
        ht_ref[...] = dinv * h_ref[...]

    return pl.pallas_call(
        body,
        grid=(n_pad // BM,),
        in_specs=[pl.BlockSpec((BM, d), lambda i: (i, 0)),
                  pl.BlockSpec((NC, BM, d), lambda i: (0, i, 0))],
        out_specs=[pl.BlockSpec((BM, d), lambda i: (i, 0)),
                   pl.BlockSpec((BM, d), lambda i: (i, 0))],
        out_shape=[jax.ShapeDtypeStruct((n_pad, d), jnp.float32),
                   jax.ShapeDtypeStruct((n_pad, d), jnp.float32)],
    )(h, degc)


def _tc_mid(agg, ht, dinv, b1):
    n_pad, d = ht.shape

    def body(ag_ref, ht_ref, di_ref, b_ref, o_ref):
        dinv = di_ref[...]
        z = dinv * (ag_ref[0] + ag_ref[1] + ht_ref[...]) + b_ref[...]
        z = jnp.maximum(z, 0.0)
        o_ref[...] = dinv * z

    return pl.pallas_call(
        body,
        grid=(n_pad // BM,),
        in_specs=[pl.BlockSpec((NC, BM, d), lambda i: (0, i, 0)),
                  pl.BlockSpec((BM, d), lambda i: (i, 0)),
                  pl.BlockSpec((BM, d), lambda i: (i, 0)),
                  pl.BlockSpec((1, d), lambda i: (0, 0))],
        out_specs=pl.BlockSpec((BM, d), lambda i: (i, 0)),
        out_shape=jax.ShapeDtypeStruct((n_pad, d), jnp.float32),
    )(agg, ht, dinv, b1)


def _tc_out(agg, ht, dinv, W2, b2, n_out):
    n_pad, d = ht.shape
    d_o = W2.shape[1]

    def body(ag_ref, ht_ref, di_ref, w_ref, b_ref, o_ref):
        pre = di_ref[...] * (ag_ref[0] + ag_ref[1] + ht_ref[...])
        acc = jnp.dot(pre, w_ref[...], preferred_element_type=jnp.float32)
        o_ref[...] = jnp.maximum(acc + b_ref[...], 0.0)

    return pl.pallas_call(
        body,
        grid=(n_pad // BM,),
        in_specs=[pl.BlockSpec((NC, BM, d), lambda i: (0, i, 0)),
                  pl.BlockSpec((BM, d), lambda i: (i, 0)),
                  pl.BlockSpec((BM, d), lambda i: (i, 0)),
                  pl.BlockSpec((d, d_o), lambda i: (0, 0)),
                  pl.BlockSpec((1, d_o), lambda i: (0, 0))],
        out_specs=pl.BlockSpec((BM, d_o), lambda i: (i, 0)),
        out_shape=jax.ShapeDtypeStruct((n_out, d_o), jnp.float32),
    )(agg, ht, dinv, W2, b2)


def kernel(x, edge_index, W1, b1, W2, b2):
    N, d_in = x.shape
    d_hid = W1.shape[1]
    d_out = W2.shape[1]
    E = edge_index.shape[1]

    kt = -(-E // (NS * CHUNK))
    kt = -(-kt // 16) * 16
    k0 = kt // 2
    k1 = kt - k0
    n_chunks = NS * kt
    e_pad = n_chunks * CHUNK
    assert n_chunks % NW == 0
    kd = n_chunks // NW

    n_acc = -(-(N + LANES) // (NS * 8)) * (NS * 8)
    n_pad = -(-max(n_acc, BM) // BM) * BM

    src = edge_index[0].astype(jnp.int32)
    dst = edge_index[1].astype(jnp.int32)
    if e_pad > E:
        fill = N + (jnp.arange(e_pad - E, dtype=jnp.int32) % (n_acc - N))
        src = jnp.concatenate([src, fill])
        dst = jnp.concatenate([dst, fill])
    src_r = src.reshape(n_chunks, CHUNK)
    dst_r = dst.reshape(n_chunks, CHUNK)
    b1r = b1.reshape(1, d_hid)
    b2r = b2.reshape(1, d_out)

    deg_fn = _make_deg_kernel(n_acc, n_pad, kd, d_hid)
    agg_fn = _make_agg_kernel(n_acc, n_pad, k0, k1, d_hid)

    degc = deg_fn(dst_r)
    h1 = _tc_matmul(x, W1, n_pad)
    ht1, dinv = _tc_scale(h1, degc)
    agg1 = agg_fn(ht1, src_r, dst_r)
    htz = _tc_mid(agg1, ht1, dinv, b1r)
    agg2 = agg_fn(htz, src_r, dst_r)
    return _tc_out(agg2, htz, dinv, W2, b2r, N)
